# baseline (device time: 67055 ns/iter reference)
import jax
import jax.numpy as jnp
from jax import lax
from jax.experimental import pallas as pl
from jax.experimental.pallas import tpu as pltpu

N_DEV = 4
B, Sq, Hq, Dh = 2, 512, 8, 64
SKV = 512
D_MODEL = 768
D_QK = Hq * Dh
SCALE = 0.125


def kernel(x, Wq, K_ext, V_ext, Wo):
    def body(x_ref, wq_ref, k_ref, v_ref, wo_ref, out_ref,
             q_ref, st_ref, l_ref, stL_ref, lL_ref, stR_ref, lR_ref,
             stD_ref, lD_ref, acc_ref, lacc_ref, ctx_ref,
             send_sems, recv_sems):
        my = lax.axis_index("i")
        left = lax.rem(my + N_DEV - 1, N_DEV)
        right = lax.rem(my + 1, N_DEV)

        wqb = wq_ref[...].astype(jnp.bfloat16)
        for b in range(B):
            q_ref[b] = lax.dot_general(
                x_ref[b].astype(jnp.bfloat16), wqb,
                (((1,), (0,)), ((), ())),
                preferred_element_type=jnp.float32,
            ).astype(jnp.bfloat16)

        off = my * SKV
        qi = lax.broadcasted_iota(jnp.int32, (Sq, SKV), 0)
        kg = lax.broadcasted_iota(jnp.int32, (Sq, SKV), 1) + off
        mask = (jnp.abs(qi - kg) <= 128) | (kg < 32) | (qi < 32)
        for b in range(B):
            for hd in range(Hq):
                sl = slice(hd * Dh, (hd + 1) * Dh)
                s = lax.dot_general(
                    q_ref[b, :, sl],
                    k_ref[b, :, hd, :].astype(jnp.bfloat16),
                    (((1,), (1,)), ((), ())),
                    preferred_element_type=jnp.float32,
                ) * SCALE
                p = jnp.where(mask, jnp.exp(s), 0.0)
                l_ref[b, :, hd:hd + 1] = jnp.sum(p, axis=1, keepdims=True)
                st_ref[b, :, sl] = lax.dot_general(
                    p.astype(jnp.bfloat16),
                    v_ref[b, :, hd, :].astype(jnp.bfloat16),
                    (((1,), (0,)), ((), ())),
                    preferred_element_type=jnp.float32,
                ).astype(jnp.bfloat16)

        barrier = pltpu.get_barrier_semaphore()
        for nbr in (left, right):
            pl.semaphore_signal(barrier, inc=1, device_id=(nbr,),
                                device_id_type=pl.DeviceIdType.MESH)
        pl.semaphore_wait(barrier, 2)

        def rdma(i, src, dst, dev):
            return pltpu.make_async_remote_copy(
                src_ref=src, dst_ref=dst,
                send_sem=send_sems.at[i], recv_sem=recv_sems.at[i],
                device_id=(dev,), device_id_type=pl.DeviceIdType.MESH,
            )

        hop1 = [
            rdma(0, st_ref, stL_ref, right),
            rdma(1, l_ref, lL_ref, right),
            rdma(2, st_ref, stR_ref, left),
            rdma(3, l_ref, lR_ref, left),
        ]
        for r in hop1:
            r.start()
        for r in hop1:
            r.wait()

        hop2 = [
            rdma(4, stL_ref.at[0:1], stD_ref.at[0:1], right),
            rdma(5, lL_ref.at[0:1], lD_ref.at[0:1], right),
            rdma(6, stR_ref.at[1:2], stD_ref.at[1:2], left),
            rdma(7, lR_ref.at[1:2], lD_ref.at[1:2], left),
        ]
        for r in hop2:
            r.start()
        for b in range(B):
            acc_ref[b] = (st_ref[b].astype(jnp.float32)
                          + stL_ref[b].astype(jnp.float32)
                          + stR_ref[b].astype(jnp.float32))
            lacc_ref[b] = l_ref[b] + lL_ref[b] + lR_ref[b]
        for r in hop2:
            r.wait()

        wob = wo_ref[...].astype(jnp.bfloat16)
        for b in range(B):
            acc = acc_ref[b] + stD_ref[b].astype(jnp.float32)
            l_tot = lacc_ref[b] + lD_ref[b]
            for hd in range(Hq):
                sl = slice(hd * Dh, (hd + 1) * Dh)
                ctx_ref[b, :, sl] = (
                    acc[:, sl] / l_tot[:, hd:hd + 1]
                ).astype(jnp.bfloat16)
            out_ref[b] = lax.dot_general(
                ctx_ref[b], wob, (((1,), (0,)), ((), ())),
                preferred_element_type=jnp.float32,
            )

    st_shape = pltpu.VMEM((B, Sq, D_QK), jnp.bfloat16)
    l_shape = pltpu.VMEM((B, Sq, Hq), jnp.float32)
    return pl.pallas_call(
        body,
        out_shape=jax.ShapeDtypeStruct((B, Sq, D_MODEL), jnp.float32),
        in_specs=[pl.BlockSpec(memory_space=pltpu.VMEM)] * 5,
        out_specs=pl.BlockSpec(memory_space=pltpu.VMEM),
        scratch_shapes=[
            pltpu.VMEM((B, Sq, D_QK), jnp.bfloat16),
            st_shape, l_shape,
            st_shape, l_shape,
            st_shape, l_shape,
            st_shape, l_shape,
            pltpu.VMEM((B, Sq, D_QK), jnp.float32),
            l_shape,
            pltpu.VMEM((B, Sq, D_QK), jnp.bfloat16),
            pltpu.SemaphoreType.DMA((8,)),
            pltpu.SemaphoreType.DMA((8,)),
        ],
        compiler_params=pltpu.CompilerParams(
            collective_id=0, vmem_limit_bytes=100 * 1024 * 1024),
    )(x, Wq, K_ext, V_ext, Wo)


# device time: 53532 ns/iter; 1.2526x vs baseline; 1.2526x over previous
import jax
import jax.numpy as jnp
from jax import lax
from jax.experimental import pallas as pl
from jax.experimental.pallas import tpu as pltpu

N_DEV = 4
B, Sq, Hq, Dh = 2, 512, 8, 64
SKV = 512
D_MODEL = 768
D_QK = Hq * Dh
D_ST = 640
SCALE = 0.125


def kernel(x, Wq, K_ext, V_ext, Wo):
    xb = x.astype(jnp.bfloat16)
    wqb = Wq.astype(jnp.bfloat16)
    wob = Wo.astype(jnp.bfloat16)
    kv = jnp.stack([
        K_ext.reshape(B, SKV, D_QK),
        V_ext.reshape(B, SKV, D_QK),
    ]).astype(jnp.bfloat16)

    def body(x_ref, wq_ref, kv_ref, wo_ref, out_ref,
             q_ref, st_ref, stL_ref, stR_ref, stD_ref,
             acc_ref, lacc_ref, ctx_ref, send_sems, recv_sems):
        my = lax.axis_index("i")
        left = lax.rem(my + N_DEV - 1, N_DEV)
        right = lax.rem(my + 1, N_DEV)

        for b in range(B):
            q_ref[b] = lax.dot_general(
                x_ref[b], wq_ref[...],
                (((1,), (0,)), ((), ())),
                preferred_element_type=jnp.float32,
            ).astype(jnp.bfloat16)

        off = my * SKV
        qi = lax.broadcasted_iota(jnp.int32, (Sq, SKV), 0)
        kg = lax.broadcasted_iota(jnp.int32, (Sq, SKV), 1) + off
        mask = (jnp.abs(qi - kg) <= 128) | (kg < 32) | (qi < 32)
        for b in range(B):
            for hd in range(Hq):
                sl = slice(hd * Dh, (hd + 1) * Dh)
                s = lax.dot_general(
                    q_ref[b, :, sl], kv_ref[0, b, :, sl],
                    (((1,), (1,)), ((), ())),
                    preferred_element_type=jnp.float32,
                ) * SCALE
                p = jnp.where(mask, jnp.exp(s), 0.0)
                st_ref[b, :, D_QK + hd:D_QK + hd + 1] = jnp.sum(
                    p, axis=1, keepdims=True).astype(jnp.bfloat16)
                st_ref[b, :, sl] = lax.dot_general(
                    p.astype(jnp.bfloat16), kv_ref[1, b, :, sl],
                    (((1,), (0,)), ((), ())),
                    preferred_element_type=jnp.float32,
                ).astype(jnp.bfloat16)

        barrier = pltpu.get_barrier_semaphore()
        for nbr in (left, right):
            pl.semaphore_signal(barrier, inc=1, device_id=(nbr,),
                                device_id_type=pl.DeviceIdType.MESH)
        pl.semaphore_wait(barrier, 2)

        def rdma(i, src, dst, dev):
            return pltpu.make_async_remote_copy(
                src_ref=src, dst_ref=dst,
                send_sem=send_sems.at[i], recv_sem=recv_sems.at[i],
                device_id=(dev,), device_id_type=pl.DeviceIdType.MESH,
            )

        hop1 = [
            rdma(0, st_ref, stL_ref, right),
            rdma(1, st_ref, stR_ref, left),
        ]
        for r in hop1:
            r.start()
        for r in hop1:
            r.wait()

        hop2 = [
            rdma(2, stL_ref.at[0:1], stD_ref.at[0:1], right),
            rdma(3, stR_ref.at[1:2], stD_ref.at[1:2], left),
        ]
        for r in hop2:
            r.start()
        for b in range(B):
            acc_ref[b] = (st_ref[b, :, :D_QK].astype(jnp.float32)
                          + stL_ref[b, :, :D_QK].astype(jnp.float32)
                          + stR_ref[b, :, :D_QK].astype(jnp.float32))
            lacc_ref[b] = (
                st_ref[b, :, D_QK:D_QK + Hq].astype(jnp.float32)
                + stL_ref[b, :, D_QK:D_QK + Hq].astype(jnp.float32)
                + stR_ref[b, :, D_QK:D_QK + Hq].astype(jnp.float32))
        for r in hop2:
            r.wait()

        for b in range(B):
            acc = acc_ref[b] + stD_ref[b, :, :D_QK].astype(jnp.float32)
            l_tot = (lacc_ref[b]
                     + stD_ref[b, :, D_QK:D_QK + Hq].astype(jnp.float32))
            for hd in range(Hq):
                sl = slice(hd * Dh, (hd + 1) * Dh)
                ctx_ref[b, :, sl] = (
                    acc[:, sl] / l_tot[:, hd:hd + 1]
                ).astype(jnp.bfloat16)
            out_ref[b] = lax.dot_general(
                ctx_ref[b], wo_ref[...], (((1,), (0,)), ((), ())),
                preferred_element_type=jnp.float32,
            )

    st_shape = pltpu.VMEM((B, Sq, D_ST), jnp.bfloat16)
    return pl.pallas_call(
        body,
        out_shape=jax.ShapeDtypeStruct((B, Sq, D_MODEL), jnp.float32),
        in_specs=[pl.BlockSpec(memory_space=pltpu.VMEM)] * 4,
        out_specs=pl.BlockSpec(memory_space=pltpu.VMEM),
        scratch_shapes=[
            pltpu.VMEM((B, Sq, D_QK), jnp.bfloat16),
            st_shape,
            st_shape,
            st_shape,
            st_shape,
            pltpu.VMEM((B, Sq, D_QK), jnp.float32),
            pltpu.VMEM((B, Sq, Hq), jnp.float32),
            pltpu.VMEM((B, Sq, D_QK), jnp.bfloat16),
            pltpu.SemaphoreType.DMA((4,)),
            pltpu.SemaphoreType.DMA((4,)),
        ],
        compiler_params=pltpu.CompilerParams(
            collective_id=0, vmem_limit_bytes=100 * 1024 * 1024),
    )(xb, wqb, kv, wob)


# device time: 52525 ns/iter; 1.2766x vs baseline; 1.0192x over previous
import jax
import jax.numpy as jnp
from jax import lax
from jax.experimental import pallas as pl
from jax.experimental.pallas import tpu as pltpu

N_DEV = 4
B, Sq, Hq, Dh = 2, 512, 8, 64
SKV = 512
D_MODEL = 768
D_QK = Hq * Dh
D_ST = 640
SCALE = 0.125


def kernel(x, Wq, K_ext, V_ext, Wo):
    xb = x.astype(jnp.bfloat16)
    wqb = Wq.astype(jnp.bfloat16)
    wob = Wo.astype(jnp.bfloat16)
    kv = jnp.stack([
        K_ext.reshape(B, SKV, D_QK),
        V_ext.reshape(B, SKV, D_QK),
    ]).astype(jnp.bfloat16)

    def body(x_ref, wq_ref, kv_ref, wo_ref, out_ref,
             q_ref, st_ref, stL_ref, stR_ref, stD_ref,
             acc_ref, lacc_ref, ctx_ref, send_sems, recv_sems):
        my = lax.axis_index("i")
        left = lax.rem(my + N_DEV - 1, N_DEV)
        right = lax.rem(my + 1, N_DEV)

        for b in range(B):
            q_ref[b] = lax.dot_general(
                x_ref[b], wq_ref[...],
                (((1,), (0,)), ((), ())),
                preferred_element_type=jnp.float32,
            ).astype(jnp.bfloat16)

        off = my * SKV
        qi = lax.broadcasted_iota(jnp.int32, (Sq, SKV), 0)
        kg = lax.broadcasted_iota(jnp.int32, (Sq, SKV), 1) + off
        mask = (jnp.abs(qi - kg) <= 128) | (kg < 32) | (qi < 32)
        HALF = Sq // 2

        def partial(r0, r1):
            for b in range(B):
                for hd in range(Hq):
                    sl = slice(hd * Dh, (hd + 1) * Dh)
                    s = lax.dot_general(
                        q_ref[b, r0:r1, sl], kv_ref[0, b, :, sl],
                        (((1,), (1,)), ((), ())),
                        preferred_element_type=jnp.float32,
                    ) * SCALE
                    p = jnp.where(mask[r0:r1], jnp.exp(s), 0.0)
                    st_ref[b, r0:r1, D_QK + hd:D_QK + hd + 1] = jnp.sum(
                        p, axis=1, keepdims=True).astype(jnp.bfloat16)
                    st_ref[b, r0:r1, sl] = lax.dot_general(
                        p.astype(jnp.bfloat16), kv_ref[1, b, :, sl],
                        (((1,), (0,)), ((), ())),
                        preferred_element_type=jnp.float32,
                    ).astype(jnp.bfloat16)

        partial(0, HALF)

        barrier = pltpu.get_barrier_semaphore()
        for nbr in (left, right):
            pl.semaphore_signal(barrier, inc=1, device_id=(nbr,),
                                device_id_type=pl.DeviceIdType.MESH)
        pl.semaphore_wait(barrier, 2)

        def rdma(i, src, dst, dev):
            return pltpu.make_async_remote_copy(
                src_ref=src, dst_ref=dst,
                send_sem=send_sems.at[i], recv_sem=recv_sems.at[i],
                device_id=(dev,), device_id_type=pl.DeviceIdType.MESH,
            )

        def merge(r0, r1):
            for b in range(B):
                acc_ref[b, r0:r1] = (
                    st_ref[b, r0:r1, :D_QK].astype(jnp.float32)
                    + stL_ref[b, r0:r1, :D_QK].astype(jnp.float32)
                    + stR_ref[b, r0:r1, :D_QK].astype(jnp.float32))
                lacc_ref[b, r0:r1] = (
                    st_ref[b, r0:r1, D_QK:D_QK + Hq].astype(jnp.float32)
                    + stL_ref[b, r0:r1, D_QK:D_QK + Hq].astype(jnp.float32)
                    + stR_ref[b, r0:r1, D_QK:D_QK + Hq].astype(jnp.float32))

        h1a = [
            rdma(0, st_ref.at[:, 0:HALF], stL_ref.at[:, 0:HALF], right),
            rdma(1, st_ref.at[:, 0:HALF], stR_ref.at[:, 0:HALF], left),
        ]
        for r in h1a:
            r.start()
        partial(HALF, Sq)
        for r in h1a:
            r.wait()

        h1b = [
            rdma(2, st_ref.at[:, HALF:Sq], stL_ref.at[:, HALF:Sq], right),
            rdma(3, st_ref.at[:, HALF:Sq], stR_ref.at[:, HALF:Sq], left),
        ]
        h2a = [
            rdma(4, stL_ref.at[0:1, 0:HALF], stD_ref.at[0:1, 0:HALF], right),
            rdma(5, stR_ref.at[1:2, 0:HALF], stD_ref.at[1:2, 0:HALF], left),
        ]
        for r in h1b + h2a:
            r.start()
        merge(0, HALF)
        for r in h1b:
            r.wait()

        h2b = [
            rdma(6, stL_ref.at[0:1, HALF:Sq], stD_ref.at[0:1, HALF:Sq],
                 right),
            rdma(7, stR_ref.at[1:2, HALF:Sq], stD_ref.at[1:2, HALF:Sq],
                 left),
        ]
        for r in h2b:
            r.start()
        merge(HALF, Sq)
        for r in h2a + h2b:
            r.wait()

        for b in range(B):
            acc = acc_ref[b] + stD_ref[b, :, :D_QK].astype(jnp.float32)
            l_tot = (lacc_ref[b]
                     + stD_ref[b, :, D_QK:D_QK + Hq].astype(jnp.float32))
            for hd in range(Hq):
                sl = slice(hd * Dh, (hd + 1) * Dh)
                ctx_ref[b, :, sl] = (
                    acc[:, sl] / l_tot[:, hd:hd + 1]
                ).astype(jnp.bfloat16)
            out_ref[b] = lax.dot_general(
                ctx_ref[b], wo_ref[...], (((1,), (0,)), ((), ())),
                preferred_element_type=jnp.float32,
            )

    st_shape = pltpu.VMEM((B, Sq, D_ST), jnp.bfloat16)
    return pl.pallas_call(
        body,
        out_shape=jax.ShapeDtypeStruct((B, Sq, D_MODEL), jnp.float32),
        in_specs=[pl.BlockSpec(memory_space=pltpu.VMEM)] * 4,
        out_specs=pl.BlockSpec(memory_space=pltpu.VMEM),
        scratch_shapes=[
            pltpu.VMEM((B, Sq, D_QK), jnp.bfloat16),
            st_shape,
            st_shape,
            st_shape,
            st_shape,
            pltpu.VMEM((B, Sq, D_QK), jnp.float32),
            pltpu.VMEM((B, Sq, Hq), jnp.float32),
            pltpu.VMEM((B, Sq, D_QK), jnp.bfloat16),
            pltpu.SemaphoreType.DMA((8,)),
            pltpu.SemaphoreType.DMA((8,)),
        ],
        compiler_params=pltpu.CompilerParams(
            collective_id=0, vmem_limit_bytes=100 * 1024 * 1024),
    )(xb, wqb, kv, wob)


# device time: 50619 ns/iter; 1.3247x vs baseline; 1.0377x over previous
import jax
import jax.numpy as jnp
from jax import lax
from jax.experimental import pallas as pl
from jax.experimental.pallas import tpu as pltpu

N_DEV = 4
B, Sq, Hq, Dh = 2, 512, 8, 64
SKV = 512
D_MODEL = 768
D_QK = Hq * Dh
D_ST = 640
HALF = Sq // 2
SCALE = 0.125


def kernel(x, Wq, K_ext, V_ext, Wo):
    xb = x.astype(jnp.bfloat16)
    wqb = Wq.astype(jnp.bfloat16)
    wob = Wo.astype(jnp.bfloat16)
    kb = K_ext.reshape(B, SKV, D_QK).astype(jnp.bfloat16)
    vb = V_ext.reshape(B, SKV, D_QK).astype(jnp.bfloat16)

    def body(x_ref, wq_ref, k_ref, v_ref, wo_ref, out_ref,
             q_ref, stA_ref, stB_ref, stLA_ref, stLB_ref,
             stRA_ref, stRB_ref, stDA_ref, stDB_ref,
             ctx_ref, send_sems, recv_sems):
        my = lax.axis_index("i")
        left = lax.rem(my + N_DEV - 1, N_DEV)
        right = lax.rem(my + 1, N_DEV)

        for b in range(B):
            q_ref[b] = lax.dot_general(
                x_ref[b], wq_ref[...],
                (((1,), (0,)), ((), ())),
                preferred_element_type=jnp.float32,
            ).astype(jnp.bfloat16)

        off = my * SKV
        qi = lax.broadcasted_iota(jnp.int32, (Sq, SKV), 0)
        kg = lax.broadcasted_iota(jnp.int32, (Sq, SKV), 1) + off
        mask = (jnp.abs(qi - kg) <= 128) | (kg < 32) | (qi < 32)

        def partial(st, r0):
            for b in range(B):
                for hd in range(Hq):
                    sl = slice(hd * Dh, (hd + 1) * Dh)
                    s = lax.dot_general(
                        q_ref[b, r0:r0 + HALF, sl], k_ref[b, :, sl],
                        (((1,), (1,)), ((), ())),
                        preferred_element_type=jnp.float32,
                    ) * SCALE
                    p = jnp.where(mask[r0:r0 + HALF], jnp.exp(s), 0.0)
                    st[b, :, D_QK + hd:D_QK + hd + 1] = jnp.sum(
                        p, axis=1, keepdims=True).astype(jnp.bfloat16)
                    st[b, :, sl] = lax.dot_general(
                        p.astype(jnp.bfloat16), v_ref[b, :, sl],
                        (((1,), (0,)), ((), ())),
                        preferred_element_type=jnp.float32,
                    ).astype(jnp.bfloat16)

        partial(stA_ref, 0)

        barrier = pltpu.get_barrier_semaphore()
        for nbr in (left, right):
            pl.semaphore_signal(barrier, inc=1, device_id=(nbr,),
                                device_id_type=pl.DeviceIdType.MESH)
        pl.semaphore_wait(barrier, 2)

        def rdma(i, src, dst, dev):
            return pltpu.make_async_remote_copy(
                src_ref=src, dst_ref=dst,
                send_sem=send_sems.at[i], recv_sem=recv_sems.at[i],
                device_id=(dev,), device_id_type=pl.DeviceIdType.MESH,
            )

        def finalize(own, L, R, D, r0):
            for b in range(B):
                a = (own[b, :, :D_QK].astype(jnp.float32)
                     + L[b, :, :D_QK].astype(jnp.float32)
                     + R[b, :, :D_QK].astype(jnp.float32)
                     + D[b, :, :D_QK].astype(jnp.float32))
                lt = (own[b, :, D_QK:D_QK + Hq].astype(jnp.float32)
                      + L[b, :, D_QK:D_QK + Hq].astype(jnp.float32)
                      + R[b, :, D_QK:D_QK + Hq].astype(jnp.float32)
                      + D[b, :, D_QK:D_QK + Hq].astype(jnp.float32))
                for hd in range(Hq):
                    sl = slice(hd * Dh, (hd + 1) * Dh)
                    ctx_ref[b, r0:r0 + HALF, sl] = (
                        a[:, sl] / lt[:, hd:hd + 1]).astype(jnp.bfloat16)
                out_ref[b, r0:r0 + HALF] = lax.dot_general(
                    ctx_ref[b, r0:r0 + HALF], wo_ref[...],
                    (((1,), (0,)), ((), ())),
                    preferred_element_type=jnp.float32,
                ).astype(jnp.bfloat16)

        h1a = [
            rdma(0, stA_ref, stLA_ref, right),
            rdma(1, stA_ref, stRA_ref, left),
        ]
        for r in h1a:
            r.start()
        partial(stB_ref, HALF)
        for r in h1a:
            r.wait()

        h1b = [
            rdma(2, stB_ref, stLB_ref, right),
            rdma(3, stB_ref, stRB_ref, left),
        ]
        h2a = [
            rdma(4, stLA_ref.at[0:1], stDA_ref.at[0:1], right),
            rdma(5, stRA_ref.at[1:2], stDA_ref.at[1:2], left),
        ]
        for r in h1b + h2a:
            r.start()
        for r in h1b:
            r.wait()

        h2b = [
            rdma(6, stLB_ref.at[0:1], stDB_ref.at[0:1], right),
            rdma(7, stRB_ref.at[1:2], stDB_ref.at[1:2], left),
        ]
        for r in h2b:
            r.start()
        for r in h2a:
            r.wait()
        finalize(stA_ref, stLA_ref, stRA_ref, stDA_ref, 0)
        for r in h2b:
            r.wait()
        finalize(stB_ref, stLB_ref, stRB_ref, stDB_ref, HALF)

    st_shape = pltpu.VMEM((B, HALF, D_ST), jnp.bfloat16)
    return pl.pallas_call(
        body,
        out_shape=jax.ShapeDtypeStruct((B, Sq, D_MODEL), jnp.bfloat16),
        in_specs=[pl.BlockSpec(memory_space=pltpu.VMEM)] * 5,
        out_specs=pl.BlockSpec(memory_space=pltpu.VMEM),
        scratch_shapes=[
            pltpu.VMEM((B, Sq, D_QK), jnp.bfloat16),
            st_shape, st_shape,
            st_shape, st_shape,
            st_shape, st_shape,
            st_shape, st_shape,
            pltpu.VMEM((B, Sq, D_QK), jnp.bfloat16),
            pltpu.SemaphoreType.DMA((8,)),
            pltpu.SemaphoreType.DMA((8,)),
        ],
        compiler_params=pltpu.CompilerParams(
            collective_id=0, vmem_limit_bytes=100 * 1024 * 1024),
    )(xb, wqb, kb, vb, wob)


# device time: 47610 ns/iter; 1.4084x vs baseline; 1.0632x over previous
import jax
import jax.numpy as jnp
from jax import lax
from jax.experimental import pallas as pl
from jax.experimental.pallas import tpu as pltpu

N_DEV = 4
B, Sq, Hq, Dh = 2, 512, 8, 64
SKV = 512
D_MODEL = 768
D_QK = Hq * Dh
D_ST = 640
SCALE = 0.125

GLO = 32
TAIL = 128
STRIP = GLO + TAIL
MID0, MID1 = GLO, Sq - TAIL
MID = MID1 - MID0
P = MID // 2


def kernel(x, Wq, K_ext, V_ext, Wo):
    xb = x.astype(jnp.bfloat16)
    wqb = Wq.astype(jnp.bfloat16)
    wob = Wo.astype(jnp.bfloat16)
    kb = K_ext.reshape(B, SKV, D_QK).astype(jnp.bfloat16)
    vb = V_ext.reshape(B, SKV, D_QK).astype(jnp.bfloat16)

    def body(x_ref, wq_ref, k_ref, v_ref, wo_ref, out_ref,
             q_ref, sO_ref, sL_ref, sR_ref, sD_ref, m_ref,
             ctx_ref, send_sems, recv_sems):
        my = lax.axis_index("i")
        left = lax.rem(my + N_DEV - 1, N_DEV)
        right = lax.rem(my + 1, N_DEV)

        for b in range(B):
            q_ref[b] = lax.dot_general(
                x_ref[b], wq_ref[...],
                (((1,), (0,)), ((), ())),
                preferred_element_type=jnp.float32,
            ).astype(jnp.bfloat16)

        off = my * SKV
        qi = lax.broadcasted_iota(jnp.int32, (Sq, SKV), 0)
        kg = lax.broadcasted_iota(jnp.int32, (Sq, SKV), 1) + off
        mask = (jnp.abs(qi - kg) <= 128) | (kg < 32) | (qi < 32)

        def state_rows(dst, dst0, qr0, qr1, msk):
            n = qr1 - qr0
            for b in range(B):
                for hd in range(Hq):
                    sl = slice(hd * Dh, (hd + 1) * Dh)
                    s = lax.dot_general(
                        q_ref[b, qr0:qr1, sl], k_ref[b, :, sl],
                        (((1,), (1,)), ((), ())),
                        preferred_element_type=jnp.float32,
                    ) * SCALE
                    p = jnp.where(msk, jnp.exp(s), 0.0)
                    dst[b, dst0:dst0 + n, D_QK + hd:D_QK + hd + 1] = jnp.sum(
                        p, axis=1, keepdims=True).astype(jnp.bfloat16)
                    dst[b, dst0:dst0 + n, sl] = lax.dot_general(
                        p.astype(jnp.bfloat16), v_ref[b, :, sl],
                        (((1,), (0,)), ((), ())),
                        preferred_element_type=jnp.float32,
                    ).astype(jnp.bfloat16)

        state_rows(sO_ref, 0, 0, GLO, mask[0:GLO])
        state_rows(sO_ref, GLO, MID1, Sq, mask[MID1:Sq])

        barrier = pltpu.get_barrier_semaphore()
        for nbr in (left, right):
            pl.semaphore_signal(barrier, inc=1, device_id=(nbr,),
                                device_id_type=pl.DeviceIdType.MESH)
        pl.semaphore_wait(barrier, 2)

        def rdma(i, src, dst, dev):
            return pltpu.make_async_remote_copy(
                src_ref=src, dst_ref=dst,
                send_sem=send_sems.at[i], recv_sem=recv_sems.at[i],
                device_id=(dev,), device_id_type=pl.DeviceIdType.MESH,
            )

        h1 = [
            rdma(0, sO_ref, sL_ref, right),
            rdma(1, sO_ref, sR_ref, left),
        ]
        for r in h1:
            r.start()

        @pl.when(my == 0)
        def _():
            state_rows(m_ref, 0, MID0, MID1, mask[MID0:MID1])

        for r in h1:
            r.wait()

        h2 = [
            rdma(2, sL_ref.at[0:1], sD_ref.at[0:1], right),
            rdma(3, sR_ref.at[1:2], sD_ref.at[1:2], left),
        ]
        for r in h2:
            r.start()

        mA = m_ref.at[:, 0:P]
        mB = m_ref.at[:, P:MID]

        @pl.when(my == 0)
        def _():
            sends = [
                rdma(4, mA, mA, right),
                rdma(9, mB, mB, left),
                rdma(6, mB, mB, right),
                rdma(8, mA, mA, left),
            ]
            for r in sends:
                r.start()
            for r in sends:
                r.wait_send()

        @pl.when(my == 1)
        def _():
            rdma(4, mA, mA, left).wait_recv()
            fwd = rdma(5, mA, mA, right)
            fwd.start()
            rdma(6, mB, mB, left).wait_recv()
            fwd.wait_send()

        @pl.when(my == 3)
        def _():
            rdma(9, mB, mB, right).wait_recv()
            fwd = rdma(7, mB, mB, left)
            fwd.start()
            rdma(8, mA, mA, right).wait_recv()
            fwd.wait_send()

        @pl.when(my == 2)
        def _():
            rdma(5, mA, mA, left).wait_recv()
            rdma(7, mB, mB, right).wait_recv()

        for r in h2:
            r.wait()

        for b in range(B):
            a = (sO_ref[b].astype(jnp.float32)
                 + sL_ref[b].astype(jnp.float32)
                 + sR_ref[b].astype(jnp.float32)
                 + sD_ref[b].astype(jnp.float32))
            mid = m_ref[b].astype(jnp.float32)
            for hd in range(Hq):
                sl = slice(hd * Dh, (hd + 1) * Dh)
                lc = slice(D_QK + hd, D_QK + hd + 1)
                ctx_ref[b, 0:GLO, sl] = (
                    a[0:GLO, sl] / a[0:GLO, lc]).astype(jnp.bfloat16)
                ctx_ref[b, MID1:Sq, sl] = (
                    a[GLO:STRIP, sl] / a[GLO:STRIP, lc]
                ).astype(jnp.bfloat16)
                ctx_ref[b, MID0:MID1, sl] = (
                    mid[:, sl] / mid[:, lc]).astype(jnp.bfloat16)
            out_ref[b] = lax.dot_general(
                ctx_ref[b], wo_ref[...], (((1,), (0,)), ((), ())),
                preferred_element_type=jnp.float32,
            ).astype(jnp.bfloat16)

    strip_shape = pltpu.VMEM((B, STRIP, D_ST), jnp.bfloat16)
    return pl.pallas_call(
        body,
        out_shape=jax.ShapeDtypeStruct((B, Sq, D_MODEL), jnp.bfloat16),
        in_specs=[pl.BlockSpec(memory_space=pltpu.VMEM)] * 5,
        out_specs=pl.BlockSpec(memory_space=pltpu.VMEM),
        scratch_shapes=[
            pltpu.VMEM((B, Sq, D_QK), jnp.bfloat16),
            strip_shape,
            strip_shape,
            strip_shape,
            strip_shape,
            pltpu.VMEM((B, MID, D_ST), jnp.bfloat16),
            pltpu.VMEM((B, Sq, D_QK), jnp.bfloat16),
            pltpu.SemaphoreType.DMA((10,)),
            pltpu.SemaphoreType.DMA((10,)),
        ],
        compiler_params=pltpu.CompilerParams(
            collective_id=0, vmem_limit_bytes=100 * 1024 * 1024),
    )(xb, wqb, kb, vb, wob)
